# Initial kernel scaffold; baseline (speedup 1.0000x reference)
import jax
import jax.numpy as jnp
from jax.experimental import pallas as pl


def kernel(x, tok_emb, pos_emb, gamma, beta):
    raise NotImplementedError("write your pallas kernel here")


# speedup vs baseline: 1.6123x; 1.6123x over previous
import functools

import jax
import jax.numpy as jnp
from jax import lax
from jax.experimental import pallas as pl
from jax.experimental.pallas import tpu as pltpu
from jax.experimental.pallas import tpu_sc as plsc

_BATCH = 4
_SEQ = 2048
_D = 1024
_B = _BATCH * _SEQ

_NC, _NS = 2, 16
_NW = _NC * _NS
_ROWS_PER_W = _B // _NW
_CHUNK = 32
_NCHUNK = _ROWS_PER_W // _CHUNK
_NBUF = 3


def _sc_gather(x_grp, tok_emb):
    mesh = plsc.VectorSubcoreMesh(core_axis_name="c", subcore_axis_name="s")

    @functools.partial(
        pl.kernel,
        mesh=mesh,
        out_type=jax.ShapeDtypeStruct((_B, _D), jnp.float32),
        scratch_types=[
            pltpu.VMEM((_NCHUNK, _CHUNK), jnp.int32),
            *[pltpu.VMEM((_CHUNK, _D), jnp.float32) for _ in range(_NBUF)],
            pltpu.SemaphoreType.DMA,
            pltpu.SemaphoreType.DMA,
        ],
    )
    def k(x_hbm, tok_hbm, out_hbm, idx_v, buf0, buf1, buf2, gsem, wsem):
        bufs = (buf0, buf1, buf2)
        wid = lax.axis_index("s") * _NC + lax.axis_index("c")
        base = wid * _ROWS_PER_W

        pltpu.sync_copy(x_hbm.at[wid], idx_v)

        def gather(c):
            return pltpu.make_async_copy(
                tok_hbm.at[idx_v.at[c]], bufs[c % _NBUF], gsem)

        def write(c):
            return pltpu.make_async_copy(
                bufs[c % _NBUF],
                out_hbm.at[pl.ds(base + c * _CHUNK, _CHUNK)],
                wsem)

        gather(0).start()
        gather(1).start()
        for c in range(_NCHUNK):
            gather(c).wait()
            write(c).start()
            if c + 2 < _NCHUNK:
                if c >= 1:
                    write(c - 1).wait()
                gather(c + 2).start()
        for c in range(max(0, _NCHUNK - 3), _NCHUNK):
            write(c).wait()

    return k(x_grp, tok_emb)


_TBLK = 256


def _tc_add_ln(g_flat, pos_emb, gamma2, beta2):
    def body(g_ref, p_ref, gam_ref, bet_ref, o_ref):
        h = g_ref[...] + p_ref[...]
        mean = jnp.mean(h, axis=-1, keepdims=True)
        cen = h - mean
        var = jnp.mean(cen * cen, axis=-1, keepdims=True)
        o_ref[...] = cen * lax.rsqrt(var + 1e-5) * gam_ref[...] + bet_ref[...]

    nper = _SEQ // _TBLK
    return pl.pallas_call(
        body,
        grid=(_B // _TBLK,),
        in_specs=[
            pl.BlockSpec((_TBLK, _D), lambda i: (i, 0)),
            pl.BlockSpec((_TBLK, _D), lambda i: (i % nper, 0)),
            pl.BlockSpec((1, _D), lambda i: (0, 0)),
            pl.BlockSpec((1, _D), lambda i: (0, 0)),
        ],
        out_specs=pl.BlockSpec((_TBLK, _D), lambda i: (i, 0)),
        out_shape=jax.ShapeDtypeStruct((_B, _D), jnp.float32),
    )(g_flat, pos_emb, gamma2, beta2)


def kernel(x, tok_emb, pos_emb, gamma, beta):
    x_grp = x.astype(jnp.int32).reshape(_NW, _NCHUNK, _CHUNK)
    g = _sc_gather(x_grp, tok_emb)
    out = _tc_add_ln(g, pos_emb, gamma.reshape(1, _D), beta.reshape(1, _D))
    return out.reshape(_BATCH, _SEQ, _D)
